# Initial kernel scaffold; baseline (speedup 1.0000x reference)
import jax
import jax.numpy as jnp
from jax.experimental import pallas as pl


def kernel(inout_labels, near_labels, neg_labels, in_embed, out_embed):
    raise NotImplementedError("write your pallas kernel here")


# speedup vs baseline: 4.6369x; 4.6369x over previous
import functools

import jax
import jax.numpy as jnp
from jax import lax
from jax.experimental import pallas as pl
from jax.experimental.pallas import tpu as pltpu
from jax.experimental.pallas import tpu_sc as plsc

_B = 16384
_D = 64
_W = 20
_N = 50
_R = _W + _N

_NC = 2
_NS = 16
_NW = _NC * _NS
_L = 16

_PER_W = _B // _NW
_C = 8
_CHUNKS = _PER_W // _C
_CR = _C * _R
_G = 112
_NG = _CR // _G


def _sc_dots(in_embed, out_embed, inout_idx, all_idx):
  mesh = plsc.VectorSubcoreMesh(core_axis_name="c", subcore_axis_name="s")

  @functools.partial(
      pl.kernel,
      out_type=jax.ShapeDtypeStruct((_B * _R,), jnp.float32),
      mesh=mesh,
      compiler_params=pltpu.CompilerParams(
          needs_layout_passes=False, use_tc_tiling_on_sc=False),
      scratch_types=[
          pltpu.VMEM((_CR,), jnp.int32),
          pltpu.VMEM((_C,), jnp.int32),
          pltpu.VMEM((_CR + _L, _D), jnp.float32),
          pltpu.VMEM((_C, _D), jnp.float32),
          pltpu.VMEM((_CR,), jnp.float32),
          pltpu.VMEM((_L, _L), jnp.float32),
          pltpu.SemaphoreType.DMA,
      ],
  )
  def k(in_hbm, out_hbm, ii_hbm, idx_hbm, dots_hbm, idx_v, ii_v, rows_v, in_v,
        dots_v, scr_v, sem):
    wid = lax.axis_index("s") * _NC + lax.axis_index("c")
    lanes = lax.iota(jnp.int32, _L)

    def chunk_body(c, _):
      base_i = wid * _PER_W + c * _C
      pltpu.sync_copy(idx_hbm.at[pl.ds(base_i * _R, _CR)], idx_v)
      pltpu.sync_copy(ii_hbm.at[pl.ds(base_i, _C)], ii_v)
      cps = [pltpu.async_copy(in_hbm.at[ii_v], in_v, sem)]
      for g in range(_NG):
        cps.append(
            pltpu.async_copy(
                out_hbm.at[idx_v.at[pl.ds(g * _G, _G)]],
                rows_v.at[pl.ds(g * _G, _G)], sem))
      for cp in cps:
        cp.wait()

      def item_body(i, _):
        x0 = in_v[i, pl.ds(0, _L)]
        x1 = in_v[i, pl.ds(_L, _L)]
        x2 = in_v[i, pl.ds(2 * _L, _L)]
        x3 = in_v[i, pl.ds(3 * _L, _L)]

        for g in range(5):
          valid = _L if g < 4 else (_R - 4 * _L)
          base_r = i * _R + g * _L
          for kk in range(valid):
            r = base_r + kk
            acc = (x0 * rows_v[r, pl.ds(0, _L)] +
                   x1 * rows_v[r, pl.ds(_L, _L)] +
                   x2 * rows_v[r, pl.ds(2 * _L, _L)] +
                   x3 * rows_v[r, pl.ds(3 * _L, _L)])
            plsc.store_scatter(scr_v, [lanes, jnp.full((_L,), kk, jnp.int32)],
                               acc)
          tot = scr_v[0, pl.ds(0, _L)]
          for s in range(1, _L):
            tot = tot + scr_v[s, pl.ds(0, _L)]
          plsc.store_scatter(dots_v, [jnp.full((_L,), base_r, jnp.int32) + lanes],
                             tot, mask=lanes < valid)
        return 0

      lax.fori_loop(0, _C, item_body, 0)
      pltpu.sync_copy(dots_v, dots_hbm.at[pl.ds(base_i * _R, _CR)])
      return 0

    lax.fori_loop(0, _CHUNKS, chunk_body, 0)

  return k(in_embed, out_embed, inout_idx, all_idx)


def _tc_loss_kernel(dots_ref, out_ref):
  x = dots_ref[...]
  col = lax.broadcasted_iota(jnp.int32, x.shape, 1)
  s = jnp.where(col < _W, x, -x)
  y = jnp.minimum(s, 0.0) - jnp.log1p(jnp.exp(-jnp.abs(s)))
  out_ref[...] = -jnp.sum(y, axis=1)


def _tc_loss(dots):
  blk = 2048
  return pl.pallas_call(
      _tc_loss_kernel,
      out_shape=jax.ShapeDtypeStruct((_B,), jnp.float32),
      grid=(_B // blk,),
      in_specs=[pl.BlockSpec((blk, _R), lambda i: (i, 0))],
      out_specs=pl.BlockSpec((blk,), lambda i: (i,)),
  )(dots)


@jax.jit
def kernel(inout_labels, near_labels, neg_labels, in_embed, out_embed):
  ii = inout_labels.astype(jnp.int32)
  all_idx = jnp.concatenate(
      [near_labels.astype(jnp.int32),
       neg_labels.astype(jnp.int32)], axis=1).reshape(-1)
  dots = _sc_dots(in_embed, out_embed, ii, all_idx)
  return _tc_loss(dots.reshape(_B, _R))


# speedup vs baseline: 5.8870x; 1.2696x over previous
import functools

import jax
import jax.numpy as jnp
from jax import lax
from jax.experimental import pallas as pl
from jax.experimental.pallas import tpu as pltpu
from jax.experimental.pallas import tpu_sc as plsc

_B = 16384
_D = 64
_W = 20
_N = 50
_R = _W + _N

_NC = 2
_NS = 16
_NW = _NC * _NS
_L = 16

_PER_W = _B // _NW
_C = 8
_CHUNKS = _PER_W // _C
_CR = _C * _R
_G = 112
_NG = _CR // _G
_IG = 128
_NIG = _PER_W // _IG


def _sc_dots(in_embed, out_embed, inout_idx, all_idx):
  mesh = plsc.VectorSubcoreMesh(core_axis_name="c", subcore_axis_name="s")

  @functools.partial(
      pl.kernel,
      out_type=jax.ShapeDtypeStruct((_B * _R,), jnp.float32),
      mesh=mesh,
      compiler_params=pltpu.CompilerParams(
          needs_layout_passes=False, use_tc_tiling_on_sc=False),
      scratch_types=[
          pltpu.VMEM((2, _CR), jnp.int32),
          pltpu.VMEM((_PER_W,), jnp.int32),
          pltpu.VMEM((2, _CR, _D), jnp.float32),
          pltpu.VMEM((_PER_W, _D), jnp.float32),
          pltpu.VMEM((2, _CR), jnp.float32),
          pltpu.VMEM((_C, _L, _L), jnp.float32),
          pltpu.SemaphoreType.DMA,
          pltpu.SemaphoreType.DMA,
          pltpu.SemaphoreType.DMA,
      ],
  )
  def k(in_hbm, out_hbm, ii_hbm, idx_hbm, dots_hbm, idx_v, ii_v, rows_v, in_v,
        dots_v, scr_v, sem_g, sem_i, sem_d):
    wid = lax.axis_index("s") * _NC + lax.axis_index("c")
    lanes = lax.iota(jnp.int32, _L)
    last = jnp.int32(_CHUNKS - 1)

    def fire_idx(c, q):
      cc = jnp.minimum(c, last)
      base_i = wid * _PER_W + cc * _C
      pltpu.async_copy(idx_hbm.at[pl.ds(base_i * _R, _CR)], idx_v.at[q], sem_i)

    def wait_idx(q):
      pltpu.make_async_copy(idx_hbm.at[pl.ds(0, _CR)], idx_v.at[q],
                            sem_i).wait()

    def fire_gathers(p):
      for g in range(_NG):
        pltpu.async_copy(
            out_hbm.at[idx_v.at[p].at[pl.ds(g * _G, _G)]],
            rows_v.at[p].at[pl.ds(g * _G, _G)], sem_g)

    def wait_gathers(p):
      for g in range(_NG):
        pltpu.make_async_copy(
            out_hbm.at[idx_v.at[p].at[pl.ds(g * _G, _G)]],
            rows_v.at[p].at[pl.ds(g * _G, _G)], sem_g).wait()

    def dots_store_wait(p):
      pltpu.make_async_copy(dots_v.at[p], dots_hbm.at[pl.ds(0, _CR)],
                            sem_d).wait()

    def compute_chunk(c, p):
      @pl.when(c >= 2)
      def _():
        dots_store_wait(p)

      @plsc.parallel_loop(0, _C, unroll=2)
      def item_body(i):
        ii = c * _C + i
        x0 = in_v[ii, pl.ds(0, _L)]
        x1 = in_v[ii, pl.ds(_L, _L)]
        x2 = in_v[ii, pl.ds(2 * _L, _L)]
        x3 = in_v[ii, pl.ds(3 * _L, _L)]
        iv = jnp.full((_L,), i, jnp.int32)

        for g in range(5):
          valid = _L if g < 4 else (_R - 4 * _L)
          base_r = i * _R + g * _L
          accs = []
          for kk in range(valid):
            r = base_r + kk
            accs.append(
                (x0 * rows_v[p, r, pl.ds(0, _L)] +
                 x1 * rows_v[p, r, pl.ds(_L, _L)]) +
                (x2 * rows_v[p, r, pl.ds(2 * _L, _L)] +
                 x3 * rows_v[p, r, pl.ds(3 * _L, _L)]))
          for kk in range(valid):
            plsc.store_scatter(
                scr_v, [iv, lanes, jnp.full((_L,), kk, jnp.int32)], accs[kk])
          tot = scr_v[i, 0, pl.ds(0, _L)]
          for s in range(1, _L):
            tot = tot + scr_v[i, s, pl.ds(0, _L)]
          plsc.store_scatter(
              dots_v.at[p], [jnp.full((_L,), base_r, jnp.int32) + lanes], tot,
              mask=lanes < valid)

      base_i = wid * _PER_W + c * _C
      pltpu.async_copy(dots_v.at[p], dots_hbm.at[pl.ds(base_i * _R, _CR)],
                       sem_d)

    pltpu.sync_copy(ii_hbm.at[pl.ds(wid * _PER_W, _PER_W)], ii_v)
    in_cps = []
    for g in range(_NIG):
      in_cps.append(
          pltpu.async_copy(in_hbm.at[ii_v.at[pl.ds(g * _IG, _IG)]],
                           in_v.at[pl.ds(g * _IG, _IG)], sem_g))

    fire_idx(jnp.int32(0), 0)
    wait_idx(0)
    for cp in in_cps:
      cp.wait()
    fire_gathers(0)
    fire_idx(jnp.int32(1), 1)

    def pair_body(t, _):
      for u in range(2):
        c = 2 * t + u
        p = u
        q = 1 - u
        wait_idx(q)
        fire_gathers(q)
        wait_gathers(p)
        fire_idx(c + 2, p)
        compute_chunk(c, p)
      return 0

    lax.fori_loop(0, _CHUNKS // 2, pair_body, 0)
    wait_idx(1)
    wait_gathers(0)
    dots_store_wait(0)
    dots_store_wait(1)

  return k(in_embed, out_embed, inout_idx, all_idx)


def _tc_loss_kernel(dots_ref, out_ref):
  x = dots_ref[...]
  col = lax.broadcasted_iota(jnp.int32, x.shape, 1)
  s = jnp.where(col < _W, x, -x)
  y = jnp.minimum(s, 0.0) - jnp.log1p(jnp.exp(-jnp.abs(s)))
  out_ref[...] = -jnp.sum(y, axis=1)


def _tc_loss(dots):
  blk = 2048
  return pl.pallas_call(
      _tc_loss_kernel,
      out_shape=jax.ShapeDtypeStruct((_B,), jnp.float32),
      grid=(_B // blk,),
      in_specs=[pl.BlockSpec((blk, _R), lambda i: (i, 0))],
      out_specs=pl.BlockSpec((blk,), lambda i: (i,)),
  )(dots)


@jax.jit
def kernel(inout_labels, near_labels, neg_labels, in_embed, out_embed):
  ii = inout_labels.astype(jnp.int32)
  all_idx = jnp.concatenate(
      [near_labels.astype(jnp.int32),
       neg_labels.astype(jnp.int32)], axis=1).reshape(-1)
  dots = _sc_dots(in_embed, out_embed, ii, all_idx)
  return _tc_loss(dots.reshape(_B, _R))
